# baseline (speedup 1.0000x reference)
import jax
import jax.numpy as jnp
from jax import lax
from jax.experimental import pallas as pl
from jax.experimental.pallas import tpu as pltpu
from jax.experimental.pallas import tpu_sc as plsc

BATCH = 16384
EMBED_DIM = 32
NUM_WORKERS = 32
ROWS_PER_WORKER = BATCH // NUM_WORKERS
CHUNK = 128
NUM_CHUNKS = ROWS_PER_WORKER // CHUNK

_MLP_BLOCK = 2048


def _sc_gather_body(user_table, item_table, u_idx, i_idx,
                    u_out, i_out,
                    uidx_v, iidx_v, urows_v, irows_v, sem):
    wid = lax.axis_index("s") * 2 + lax.axis_index("c")
    base = wid * ROWS_PER_WORKER

    pltpu.sync_copy(u_idx.at[wid], uidx_v)
    pltpu.sync_copy(i_idx.at[wid], iidx_v)

    copies = []
    for j in range(NUM_CHUNKS):
        copies.append(pltpu.async_copy(
            user_table.at[uidx_v.at[j]],
            urows_v.at[pl.ds(j * CHUNK, CHUNK)], sem))
    for j in range(NUM_CHUNKS):
        copies.append(pltpu.async_copy(
            item_table.at[iidx_v.at[j]],
            irows_v.at[pl.ds(j * CHUNK, CHUNK)], sem))
    for c in copies:
        c.wait()

    pltpu.sync_copy(urows_v, u_out.at[pl.ds(base, ROWS_PER_WORKER)])
    pltpu.sync_copy(irows_v, i_out.at[pl.ds(base, ROWS_PER_WORKER)])


@jax.jit
def _sc_gather(user_table, item_table, u_idx, i_idx):
    mesh = plsc.VectorSubcoreMesh(core_axis_name="c", subcore_axis_name="s")
    fn = pl.kernel(
        _sc_gather_body,
        out_type=[
            jax.ShapeDtypeStruct((BATCH, EMBED_DIM), jnp.float32),
            jax.ShapeDtypeStruct((BATCH, EMBED_DIM), jnp.float32),
        ],
        mesh=mesh,
        scratch_types=[
            pltpu.VMEM((NUM_CHUNKS, CHUNK), jnp.int32),
            pltpu.VMEM((NUM_CHUNKS, CHUNK), jnp.int32),
            pltpu.VMEM((ROWS_PER_WORKER, EMBED_DIM), jnp.float32),
            pltpu.VMEM((ROWS_PER_WORKER, EMBED_DIM), jnp.float32),
            pltpu.SemaphoreType.DMA,
        ],
        compiler_params=pltpu.CompilerParams(use_tc_tiling_on_sc=False),
    )
    return fn(user_table, item_table, u_idx, i_idx)


def _mlp_body(u_ref, v_ref, w1u_ref, w1i_ref, b1_ref, w2_ref, b2_ref,
              w3t_ref, b3_ref, out_ref):
    x1 = (jnp.dot(u_ref[...], w1u_ref[...], preferred_element_type=jnp.float32)
          + jnp.dot(v_ref[...], w1i_ref[...], preferred_element_type=jnp.float32)
          + b1_ref[...])
    h1 = jnp.maximum(x1, 0.0)
    h2 = jnp.maximum(
        jnp.dot(h1, w2_ref[...], preferred_element_type=jnp.float32)
        + b2_ref[...], 0.0)
    pred = jnp.sum(h2 * w3t_ref[...], axis=1, keepdims=True) + b3_ref[...]
    out_ref[...] = pred


@jax.jit
def _mlp(u, v, W1u, W1i, b1, W2, b2, W3t, b3):
    grid = (BATCH // _MLP_BLOCK,)
    full = lambda i: (0, 0)
    return pl.pallas_call(
        _mlp_body,
        grid=grid,
        in_specs=[
            pl.BlockSpec((_MLP_BLOCK, EMBED_DIM), lambda i: (i, 0)),
            pl.BlockSpec((_MLP_BLOCK, EMBED_DIM), lambda i: (i, 0)),
            pl.BlockSpec((EMBED_DIM, 64), full),
            pl.BlockSpec((EMBED_DIM, 64), full),
            pl.BlockSpec((1, 64), full),
            pl.BlockSpec((64, 32), full),
            pl.BlockSpec((1, 32), full),
            pl.BlockSpec((1, 32), full),
            pl.BlockSpec((1, 1), full),
        ],
        out_specs=pl.BlockSpec((_MLP_BLOCK, 1), lambda i: (i, 0)),
        out_shape=jax.ShapeDtypeStruct((BATCH, 1), jnp.float32),
    )(u, v, W1u, W1i, b1, W2, b2, W3t, b3)


def kernel(user_indices, item_indices, user_table, item_table,
           W1, b1, W2, b2, W3, b3):
    u_idx = user_indices.astype(jnp.int32).reshape(NUM_WORKERS, NUM_CHUNKS, CHUNK)
    i_idx = item_indices.astype(jnp.int32).reshape(NUM_WORKERS, NUM_CHUNKS, CHUNK)
    u, v = _sc_gather(user_table, item_table, u_idx, i_idx)
    W1u = W1[:EMBED_DIM, :]
    W1i = W1[EMBED_DIM:, :]
    return _mlp(u, v, W1u, W1i, b1.reshape(1, 64), W2, b2.reshape(1, 32),
                W3.reshape(1, 32), b3.reshape(1, 1))
